# baseline (speedup 1.0000x reference)
import jax
import jax.numpy as jnp
from jax import lax
from jax.experimental import pallas as pl
from jax.experimental.pallas import tpu as pltpu
from jax.experimental.pallas import tpu_sc as plsc

VOCAB = 2495767
EMBED = 64
BATCH = 16384
NNEG = 20
NPAIR = NNEG + 1
SW = 32
NW = 32
BPW = BATCH // NW
GATHER = 128
CHUNK = 128
GROUP = 32
NGROUP = CHUNK // GROUP
NCHUNK = BPW // CHUNK


def _sc_body(t_idx_hbm, c_idx_hbm, n_idx_hbm, temb, cemb, out_hbm,
             t_idx_v, c_idx_v, n_idx_v, trows, crows, nrows, scores_v, sem):
    wid = lax.axis_index("s") * 2 + lax.axis_index("c")
    lane = lax.iota(jnp.int32, 16)

    pltpu.sync_copy(t_idx_hbm.at[wid], t_idx_v)
    pltpu.sync_copy(c_idx_hbm.at[wid], c_idx_v)
    pltpu.sync_copy(n_idx_hbm.at[wid], n_idx_v)

    for chunk in range(NCHUNK):
        ct = pltpu.async_copy(temb.at[t_idx_v.at[chunk]], trows, sem)
        cc = pltpu.async_copy(cemb.at[c_idx_v.at[chunk]], crows, sem)
        ct.wait()
        cc.wait()
        for g in range(NGROUP):
            waits = []
            for j in range(5):
                row = chunk * (NGROUP * 5) + g * 5 + j
                waits.append(pltpu.async_copy(
                    cemb.at[n_idx_v.at[row]],
                    nrows.at[pl.ds(j * GATHER, GATHER)], sem))
            for w in waits:
                w.wait()

            def body(b, carry, g=g, chunk=chunk):
                row_t = g * GROUP + b
                t0 = trows[row_t, pl.ds(0, 16)]
                t1 = trows[row_t, pl.ds(16, 16)]
                t2 = trows[row_t, pl.ds(32, 16)]
                t3 = trows[row_t, pl.ds(48, 16)]
                acc = (t0 * crows[row_t, pl.ds(0, 16)]
                       + t1 * crows[row_t, pl.ds(16, 16)]
                       + t2 * crows[row_t, pl.ds(32, 16)]
                       + t3 * crows[row_t, pl.ds(48, 16)])
                lo = jnp.where(lane == 0, jnp.sum(acc), 0.0)
                hi = jnp.zeros((16,), jnp.float32)
                for n in range(NNEG):
                    rn = b * NNEG + n
                    acc = (t0 * nrows[rn, pl.ds(0, 16)]
                           + t1 * nrows[rn, pl.ds(16, 16)]
                           + t2 * nrows[rn, pl.ds(32, 16)]
                           + t3 * nrows[rn, pl.ds(48, 16)])
                    s = -jnp.sum(acc)
                    if n < 15:
                        lo = jnp.where(lane == n + 1, s, lo)
                    else:
                        hi = jnp.where(lane == n - 15, s, hi)
                row_s = chunk * CHUNK + g * GROUP + b
                scores_v[row_s, pl.ds(0, 16)] = lo
                scores_v[row_s, pl.ds(16, 16)] = hi
                return carry

            lax.fori_loop(0, GROUP, body, 0)

    pltpu.sync_copy(scores_v, out_hbm.at[pl.ds(wid * BPW, BPW)])


_sc_scores = pl.kernel(
    _sc_body,
    out_type=jax.ShapeDtypeStruct((BATCH, SW), jnp.float32),
    mesh=plsc.VectorSubcoreMesh(core_axis_name="c", subcore_axis_name="s"),
    compiler_params=pltpu.CompilerParams(
        needs_layout_passes=False, use_tc_tiling_on_sc=False),
    scratch_types=[
        pltpu.VMEM((NCHUNK, GATHER), jnp.int32),
        pltpu.VMEM((NCHUNK, GATHER), jnp.int32),
        pltpu.VMEM((BPW * NNEG // GATHER, GATHER), jnp.int32),
        pltpu.VMEM((CHUNK, EMBED), jnp.float32),
        pltpu.VMEM((CHUNK, EMBED), jnp.float32),
        pltpu.VMEM((GROUP * NNEG, EMBED), jnp.float32),
        pltpu.VMEM((BPW, SW), jnp.float32),
        pltpu.SemaphoreType.DMA,
    ],
)


def _loss_body(s_ref, o_ref):
    x = s_ref[...]
    col = lax.broadcasted_iota(jnp.int32, x.shape, 1)
    valid = (col % SW) < NPAIR
    y = jnp.where(valid, jax.nn.log_sigmoid(x), 0.0)
    o_ref[0, 0] = -jnp.sum(y) / BATCH


_tc_loss = pl.pallas_call(
    _loss_body,
    out_shape=jax.ShapeDtypeStruct((1, 1), jnp.float32),
    out_specs=pl.BlockSpec(memory_space=pltpu.SMEM),
)


@jax.jit
def kernel(target, context, negatives, target_emb, context_emb):
    t_idx = target.astype(jnp.int32).reshape(NW, NCHUNK, GATHER)
    c_idx = context.astype(jnp.int32).reshape(NW, NCHUNK, GATHER)
    n_idx = negatives.astype(jnp.int32).reshape(
        NW, BPW * NNEG // GATHER, GATHER)
    scores = _sc_scores(t_idx, c_idx, n_idx, target_emb, context_emb)
    loss = _tc_loss(scores.reshape(BATCH * SW // 128, 128))
    return loss[0, 0]


# speedup vs baseline: 1.0004x; 1.0004x over previous
import jax
import jax.numpy as jnp
from jax import lax
from jax.experimental import pallas as pl
from jax.experimental.pallas import tpu as pltpu
from jax.experimental.pallas import tpu_sc as plsc

VOCAB = 2495767
EMBED = 64
BATCH = 16384
NNEG = 20
NPAIR = NNEG + 1
SW = 32
NW = 32
BPW = BATCH // NW
CHUNK = 128
GROUP = 32
NGROUP = CHUNK // GROUP
NCHUNK = BPW // CHUNK


def _sc_body(tc_idx_hbm, n_idx_hbm, temb_flat, cemb_flat, out_hbm,
             tc_idx_v, n_idx_v, trows, crows, nrows, scores_v, sem):
    wid = lax.axis_index("s") * 2 + lax.axis_index("c")
    lane = lax.iota(jnp.int32, 16)

    pltpu.sync_copy(tc_idx_hbm.at[wid], tc_idx_v)
    pltpu.sync_copy(n_idx_hbm.at[wid], n_idx_v)

    for chunk in range(NCHUNK):
        def fire_t(i, carry, chunk=chunk):
            iv = tc_idx_v[chunk, pl.ds(i * 16, 16)]
            for j in range(16):
                off = iv[j] * EMBED
                pltpu.async_copy(temb_flat.at[pl.ds(off, EMBED)],
                                 trows.at[pl.ds((i * 16 + j) * EMBED, EMBED)],
                                 sem)
            return carry

        def fire_c(i, carry, chunk=chunk):
            iv = tc_idx_v[NCHUNK + chunk, pl.ds(i * 16, 16)]
            for j in range(16):
                off = iv[j] * EMBED
                pltpu.async_copy(cemb_flat.at[pl.ds(off, EMBED)],
                                 crows.at[pl.ds((i * 16 + j) * EMBED, EMBED)],
                                 sem)
            return carry

        lax.fori_loop(0, CHUNK // 16, fire_t, 0)
        lax.fori_loop(0, CHUNK // 16, fire_c, 0)

        for g in range(NGROUP):
            def fire_n(i, carry, chunk=chunk, g=g):
                slot = chunk * (NGROUP * 5) + g * 5 + (i >> 3)
                iv = n_idx_v[slot, pl.ds((i & 7) * 16, 16)]
                for j in range(16):
                    off = iv[j] * EMBED
                    pltpu.async_copy(cemb_flat.at[pl.ds(off, EMBED)],
                                     nrows.at[pl.ds((i * 16 + j) * EMBED,
                                                    EMBED)], sem)
                return carry

            lax.fori_loop(0, GROUP * NNEG // 16, fire_n, 0)

            if g == 0:
                pltpu.make_async_copy(
                    temb_flat.at[pl.ds(0, CHUNK * EMBED)], trows, sem).wait()
                pltpu.make_async_copy(
                    temb_flat.at[pl.ds(0, CHUNK * EMBED)], crows, sem).wait()
            pltpu.make_async_copy(
                temb_flat.at[pl.ds(0, GROUP * NNEG * EMBED)], nrows,
                sem).wait()

            def body(b, carry, g=g, chunk=chunk):
                ot = (g * GROUP + b) * EMBED
                t0 = trows[pl.ds(ot, 16)]
                t1 = trows[pl.ds(ot + 16, 16)]
                t2 = trows[pl.ds(ot + 32, 16)]
                t3 = trows[pl.ds(ot + 48, 16)]
                acc = (t0 * crows[pl.ds(ot, 16)]
                       + t1 * crows[pl.ds(ot + 16, 16)]
                       + t2 * crows[pl.ds(ot + 32, 16)]
                       + t3 * crows[pl.ds(ot + 48, 16)])
                lo = jnp.where(lane == 0, jnp.sum(acc), 0.0)
                hi = jnp.zeros((16,), jnp.float32)
                for n in range(NNEG):
                    on = (b * NNEG + n) * EMBED
                    acc = (t0 * nrows[pl.ds(on, 16)]
                           + t1 * nrows[pl.ds(on + 16, 16)]
                           + t2 * nrows[pl.ds(on + 32, 16)]
                           + t3 * nrows[pl.ds(on + 48, 16)])
                    s = -jnp.sum(acc)
                    if n < 15:
                        lo = jnp.where(lane == n + 1, s, lo)
                    else:
                        hi = jnp.where(lane == n - 15, s, hi)
                row_s = chunk * CHUNK + g * GROUP + b
                scores_v[row_s, pl.ds(0, 16)] = lo
                scores_v[row_s, pl.ds(16, 16)] = hi
                return carry

            lax.fori_loop(0, GROUP, body, 0)

    pltpu.sync_copy(scores_v, out_hbm.at[pl.ds(wid * BPW, BPW)])


_sc_scores = pl.kernel(
    _sc_body,
    out_type=jax.ShapeDtypeStruct((BATCH, SW), jnp.float32),
    mesh=plsc.VectorSubcoreMesh(core_axis_name="c", subcore_axis_name="s"),
    compiler_params=pltpu.CompilerParams(
        needs_layout_passes=False, use_tc_tiling_on_sc=False),
    scratch_types=[
        pltpu.VMEM((2 * NCHUNK, 128), jnp.int32),
        pltpu.VMEM((BPW * NNEG // 128, 128), jnp.int32),
        pltpu.VMEM((CHUNK * EMBED,), jnp.float32),
        pltpu.VMEM((CHUNK * EMBED,), jnp.float32),
        pltpu.VMEM((GROUP * NNEG * EMBED,), jnp.float32),
        pltpu.VMEM((BPW, SW), jnp.float32),
        pltpu.SemaphoreType.DMA,
    ],
)


def _loss_body(s_ref, o_ref):
    x = s_ref[...]
    col = lax.broadcasted_iota(jnp.int32, x.shape, 1)
    valid = (col % SW) < NPAIR
    y = jnp.where(valid, jax.nn.log_sigmoid(x), 0.0)
    o_ref[0, 0] = -jnp.sum(y) / BATCH


_tc_loss = pl.pallas_call(
    _loss_body,
    out_shape=jax.ShapeDtypeStruct((1, 1), jnp.float32),
    out_specs=pl.BlockSpec(memory_space=pltpu.SMEM),
)


@jax.jit
def kernel(target, context, negatives, target_emb, context_emb):
    tc_idx = jnp.concatenate(
        [target.astype(jnp.int32).reshape(NW, NCHUNK, 128),
         context.astype(jnp.int32).reshape(NW, NCHUNK, 128)], axis=1)
    n_idx = negatives.astype(jnp.int32).reshape(NW, BPW * NNEG // 128, 128)
    scores = _sc_scores(tc_idx, n_idx,
                        target_emb.reshape(-1), context_emb.reshape(-1))
    loss = _tc_loss(scores.reshape(BATCH * SW // 128, 128))
    return loss[0, 0]


# speedup vs baseline: 1.6109x; 1.6103x over previous
import jax
import jax.numpy as jnp
from jax import lax
from jax.experimental import pallas as pl
from jax.experimental.pallas import tpu as pltpu
from jax.experimental.pallas import tpu_sc as plsc

VOCAB = 2495767
EMBED = 64
BATCH = 16384
NNEG = 20
NPAIR = NNEG + 1
SW = 32
NW = 32
BPW = BATCH // NW
TRING = 2
HALF = BPW // 2


def _sc_body(tc_idx_hbm, n_idx_hbm, temb_t, cemb, out_hbm,
             tc_idx_v, nv, tstage, trows, cstage, scores_v, s_tv, s_cv, sem):
    wid = lax.axis_index("s") * 2 + lax.axis_index("c")
    lane = lax.iota(jnp.int32, 16)

    pltpu.sync_copy(tc_idx_hbm.at[wid], tc_idx_v)
    pltpu.sync_copy(n_idx_hbm.at[wid], nv)

    def stage_idx(i, carry):
        ivt = tc_idx_v[i >> 3, pl.ds((i & 7) * 16, 16)]
        ivc = tc_idx_v[4 + (i >> 3), pl.ds((i & 7) * 16, 16)]
        for j in range(16):
            s_tv[i * 16 + j] = ivt[j]
            s_cv[i * 16 + j] = ivc[j]
        return carry

    lax.fori_loop(0, BPW // 16, stage_idx, 0)

    for h in range(2):
        base = h * HALF

        def phase_t(k, carry, base=base):
            @pl.when(k >= TRING)
            def _drain_extract():
                m = k - TRING
                q = m & (TRING - 1)
                pltpu.make_async_copy(temb_t.at[:, pl.ds(0, 128)],
                                      tstage.at[pl.ds(0, 64), :], sem).wait()
                c = s_tv[base + m] & 127
                cvec = jnp.full((16,), 0, jnp.int32) + c
                for u in range(4):
                    rows = q * 64 + u * 16 + lane
                    tg = plsc.load_gather(tstage, [rows, cvec])
                    trows[pl.ds(m * EMBED + u * 16, 16)] = tg

            @pl.when(k < HALF)
            def _fire():
                q = k & (TRING - 1)
                v = s_tv[base + k]
                pltpu.async_copy(temb_t.at[:, pl.ds((v >> 7) * 128, 128)],
                                 tstage.at[pl.ds(q * 64, 64), :], sem)

            return carry

        lax.fori_loop(0, HALF + TRING, phase_t, 0)

        def phase_c(b, carry, base=base):
            @pl.when(b < HALF)
            def _fire():
                s = b & 1
                g = base + b
                vc = s_cv[g]
                pltpu.async_copy(cemb.at[pl.ds((vc >> 3) * 8, 8), :],
                                 cstage.at[pl.ds(s * NPAIR * 8, 8), :], sem)
                iv1 = nv[g >> 2, pl.ds((g & 3) * 32, 16)]
                iv2 = nv[g >> 2, pl.ds((g & 3) * 32 + 16, 16)]
                for n in range(NNEG):
                    vn = iv1[n] if n < 16 else iv2[n - 16]
                    pltpu.async_copy(
                        cemb.at[pl.ds((vn >> 3) * 8, 8), :],
                        cstage.at[pl.ds((s * NPAIR + 1 + n) * 8, 8), :], sem)

            @pl.when(b > 0)
            def _drain_compute():
                m = b - 1
                g = base + m
                s = m & 1
                pltpu.make_async_copy(cemb.at[pl.ds(0, NPAIR * 8), :],
                                      cstage.at[pl.ds(0, NPAIR * 8), :],
                                      sem).wait()
                t0 = trows[pl.ds(m * EMBED, 16)]
                t1 = trows[pl.ds(m * EMBED + 16, 16)]
                t2 = trows[pl.ds(m * EMBED + 32, 16)]
                t3 = trows[pl.ds(m * EMBED + 48, 16)]
                rc = s * NPAIR * 8 + (s_cv[g] & 7)
                acc = (t0 * cstage[rc, pl.ds(0, 16)]
                       + t1 * cstage[rc, pl.ds(16, 16)]
                       + t2 * cstage[rc, pl.ds(32, 16)]
                       + t3 * cstage[rc, pl.ds(48, 16)])
                lo = jnp.where(lane == 0, jnp.sum(acc), 0.0)
                hi = jnp.zeros((16,), jnp.float32)
                iv1 = nv[g >> 2, pl.ds((g & 3) * 32, 16)]
                iv2 = nv[g >> 2, pl.ds((g & 3) * 32 + 16, 16)]
                for n in range(NNEG):
                    vn = iv1[n] if n < 16 else iv2[n - 16]
                    rn = (s * NPAIR + 1 + n) * 8 + (vn & 7)
                    acc = (t0 * cstage[rn, pl.ds(0, 16)]
                           + t1 * cstage[rn, pl.ds(16, 16)]
                           + t2 * cstage[rn, pl.ds(32, 16)]
                           + t3 * cstage[rn, pl.ds(48, 16)])
                    sneg = -jnp.sum(acc)
                    if n < 15:
                        lo = jnp.where(lane == n + 1, sneg, lo)
                    else:
                        hi = jnp.where(lane == n - 15, sneg, hi)
                scores_v[pl.ds(m * SW, 16)] = lo
                scores_v[pl.ds(m * SW + 16, 16)] = hi

            return carry

        lax.fori_loop(0, HALF + 1, phase_c, 0)

        pltpu.sync_copy(
            scores_v,
            out_hbm.at[pl.ds((wid * BPW + base) * SW, HALF * SW)])


_sc_scores = pl.kernel(
    _sc_body,
    out_type=jax.ShapeDtypeStruct((BATCH * SW,), jnp.float32),
    mesh=plsc.VectorSubcoreMesh(core_axis_name="c", subcore_axis_name="s"),
    compiler_params=pltpu.CompilerParams(
        needs_layout_passes=False, use_tc_tiling_on_sc=True),
    scratch_types=[
        pltpu.VMEM((8, 128), jnp.int32),
        pltpu.VMEM((BPW // 4, 128), jnp.int32),
        pltpu.VMEM((TRING * 64, 128), jnp.float32),
        pltpu.VMEM((HALF * EMBED,), jnp.float32),
        pltpu.VMEM((2 * NPAIR * 8, EMBED), jnp.float32),
        pltpu.VMEM((HALF * SW,), jnp.float32),
        pltpu.SMEM((BPW,), jnp.int32),
        pltpu.SMEM((BPW,), jnp.int32),
        pltpu.SemaphoreType.DMA,
    ],
)


def _loss_body(s_ref, o_ref):
    x = s_ref[...]
    col = lax.broadcasted_iota(jnp.int32, x.shape, 1)
    valid = (col % SW) < NPAIR
    y = jnp.where(valid, jax.nn.log_sigmoid(x), 0.0)
    o_ref[0, 0] = -jnp.sum(y) / BATCH


_tc_loss = pl.pallas_call(
    _loss_body,
    out_shape=jax.ShapeDtypeStruct((1, 1), jnp.float32),
    out_specs=pl.BlockSpec(memory_space=pltpu.SMEM),
)


@jax.jit
def kernel(target, context, negatives, target_emb, context_emb):
    tc_idx = jnp.concatenate(
        [target.astype(jnp.int32).reshape(NW, 4, 128),
         context.astype(jnp.int32).reshape(NW, 4, 128)], axis=1)
    n_idx = jnp.pad(negatives.astype(jnp.int32), ((0, 0), (0, SW - NNEG))
                    ).reshape(NW, BPW // 4, 128)
    scores = _sc_scores(tc_idx, n_idx, target_emb.T, context_emb)
    loss = _tc_loss(scores.reshape(BATCH * SW // 128, 128))
    return loss[0, 0]
